# baseline (speedup 1.0000x reference)
import functools

import jax
import jax.numpy as jnp
from jax import lax
from jax.experimental import pallas as pl
from jax.experimental.pallas import tpu as pltpu
from jax.experimental.pallas import tpu_sc as plsc

B = 16384
NFLD = 12
F_IDX = 9
F_CONT = 3
NC = 2
NS = 16
L = 16
NW = NC * NS
BPW = B // NW
NX = BPW * NFLD
NI = BPW * F_IDX
GROUPS = BPW // L

_mesh = plsc.VectorSubcoreMesh(core_axis_name="c", subcore_axis_name="s")


@functools.partial(
    pl.kernel,
    mesh=_mesh,
    compiler_params=pltpu.CompilerParams(needs_layout_passes=False),
    out_type=jax.ShapeDtypeStruct((B,), jnp.float32),
    scratch_types=[
        pltpu.VMEM((NX,), jnp.int32),
        pltpu.VMEM((NI,), jnp.int32),
        pltpu.VMEM((NI,), jnp.float32),
        pltpu.VMEM((BPW * F_CONT,), jnp.float32),
        pltpu.VMEM((BPW,), jnp.float32),
        pltpu.VMEM((L,), jnp.float32),
        pltpu.SemaphoreType.DMA,
    ],
)
def _fl_kernel(x_hbm, table_hbm, bias_hbm, out_hbm,
               xv, idx_v, vals_v, cont_v, out_v, bias_v, sem):
    wid = lax.axis_index("s") * NC + lax.axis_index("c")
    pltpu.sync_copy(x_hbm.at[pl.ds(wid * NX, NX)], xv)
    pltpu.sync_copy(bias_hbm, bias_v.at[pl.ds(0, 1)])
    lanes = lax.iota(jnp.int32, L)
    zeros = lanes * 0
    for g in range(GROUPS):
        pos0 = lanes * NFLD + (g * L * NFLD)
        for f in range(F_IDX):
            idx_v[pl.ds(f * BPW + g * L, L)] = plsc.load_gather(xv, [pos0 + f])
    copy = pltpu.async_copy(table_hbm.at[idx_v], vals_v, sem)
    for g in range(GROUPS):
        pos0 = lanes * NFLD + (g * L * NFLD)
        for k in range(F_CONT):
            cont_v[pl.ds(k * BPW + g * L, L)] = (
                plsc.load_gather(xv, [pos0 + (F_IDX + k)]).astype(jnp.float32))
    bv = plsc.load_gather(bias_v, [zeros])
    copy.wait()
    for g in range(GROUPS):
        o = g * L
        acc = bv
        for f in range(6):
            acc = acc + vals_v[pl.ds(f * BPW + o, L)]
        for k in range(F_CONT):
            acc = acc + (vals_v[pl.ds((6 + k) * BPW + o, L)]
                         * cont_v[pl.ds(k * BPW + o, L)])
        out_v[pl.ds(o, L)] = acc
    pltpu.sync_copy(out_v, out_hbm.at[pl.ds(wid * BPW, BPW)])


def kernel(x, fc_weight, bias):
    x_flat = x.astype(jnp.int32).reshape(-1)
    table = fc_weight.reshape(-1).astype(jnp.float32)
    out = _fl_kernel(x_flat, table, bias.astype(jnp.float32))
    return out.reshape(B, 1)


# speedup vs baseline: 2.5671x; 2.5671x over previous
import functools

import jax
import jax.numpy as jnp
from jax import lax
from jax.experimental import pallas as pl
from jax.experimental.pallas import tpu as pltpu
from jax.experimental.pallas import tpu_sc as plsc

B = 16384
NFLD = 12
F_IDX = 9
F_CONT = 3
NC = 2
NS = 16
L = 16
NW = NC * NS
BPW = B // NW
GROUPS = BPW // L

_mesh = plsc.VectorSubcoreMesh(core_axis_name="c", subcore_axis_name="s")


@functools.partial(
    pl.kernel,
    mesh=_mesh,
    compiler_params=pltpu.CompilerParams(needs_layout_passes=False),
    out_type=jax.ShapeDtypeStruct((B,), jnp.float32),
    scratch_types=[
        pltpu.VMEM((1, NFLD * BPW), jnp.int32),
        pltpu.VMEM((1, F_IDX * BPW), jnp.float32),
        pltpu.VMEM((BPW,), jnp.float32),
        pltpu.VMEM((L,), jnp.float32),
        pltpu.SemaphoreType.DMA,
    ],
)
def _fl_kernel(xt_hbm, table_hbm, bias_hbm, out_hbm,
               xv, vals_v, out_v, bias_v, sem):
    wid = lax.axis_index("s") * NC + lax.axis_index("c")
    base = wid * BPW
    for f in range(NFLD):
        pltpu.sync_copy(xt_hbm.at[pl.ds(f, 1), pl.ds(base, BPW)],
                        xv.at[:, pl.ds(f * BPW, BPW)])
    pltpu.sync_copy(bias_hbm, bias_v.at[pl.ds(0, 1)])
    copies = [
        pltpu.async_copy(table_hbm.at[xv.at[:, pl.ds(f * BPW, BPW)]],
                         vals_v.at[:, pl.ds(f * BPW, BPW)], sem)
        for f in range(F_IDX)
    ]
    lanes = lax.iota(jnp.int32, L)
    bv = plsc.load_gather(bias_v, [lanes * 0])
    for c in copies:
        c.wait()
    for g in range(GROUPS):
        o = g * L
        acc = bv
        for f in range(6):
            acc = acc + vals_v[0, pl.ds(f * BPW + o, L)]
        for k in range(F_CONT):
            acc = acc + (vals_v[0, pl.ds((6 + k) * BPW + o, L)]
                         * xv[0, pl.ds((F_IDX + k) * BPW + o, L)].astype(jnp.float32))
        out_v[pl.ds(o, L)] = acc
    pltpu.sync_copy(out_v, out_hbm.at[pl.ds(base, BPW)])


def kernel(x, fc_weight, bias):
    xt = x.astype(jnp.int32).T
    out = _fl_kernel(xt, fc_weight.astype(jnp.float32).T,
                     bias.astype(jnp.float32))
    return out.reshape(B, 1)


# speedup vs baseline: 3.0605x; 1.1922x over previous
import functools

import jax
import jax.numpy as jnp
from jax import lax
from jax.experimental import pallas as pl
from jax.experimental.pallas import tpu as pltpu
from jax.experimental.pallas import tpu_sc as plsc

B = 16384
NFLD = 12
F_IDX = 9
F_CONT = 3
NC = 2
NS = 16
L = 16
NW = NC * NS
BPW = B // NW
GROUPS = BPW // L

_mesh = plsc.VectorSubcoreMesh(core_axis_name="c", subcore_axis_name="s")


@functools.partial(
    pl.kernel,
    mesh=_mesh,
    compiler_params=pltpu.CompilerParams(needs_layout_passes=False),
    out_type=jax.ShapeDtypeStruct((B,), jnp.float32),
    scratch_types=[
        pltpu.VMEM((1, NFLD * BPW), jnp.int32),
        pltpu.VMEM((1, F_IDX * BPW), jnp.float32),
    pltpu.VMEM((BPW,), jnp.float32),
        pltpu.VMEM((L,), jnp.float32),
        pltpu.SemaphoreType.DMA,
        [pltpu.SemaphoreType.DMA] * NFLD,
    ],
)
def _fl_kernel(xt_hbm, table_hbm, bias_hbm, out_hbm,
               xv, vals_v, out_v, bias_v, sem, row_sems):
    wid = lax.axis_index("s") * NC + lax.axis_index("c")
    base = wid * BPW
    row_copies = [
        pltpu.async_copy(xt_hbm.at[pl.ds(f, 1), pl.ds(base, BPW)],
                         xv.at[:, pl.ds(f * BPW, BPW)], row_sems[f])
        for f in range(NFLD)
    ]
    pltpu.sync_copy(bias_hbm, bias_v.at[pl.ds(0, 1)])
    copies = []
    for f in range(F_IDX):
        row_copies[f].wait()
        copies.append(
            pltpu.async_copy(table_hbm.at[xv.at[:, pl.ds(f * BPW, BPW)]],
                             vals_v.at[:, pl.ds(f * BPW, BPW)], sem))
    for f in range(F_IDX, NFLD):
        row_copies[f].wait()
    lanes = lax.iota(jnp.int32, L)
    bv = plsc.load_gather(bias_v, [lanes * 0])
    for c in copies:
        c.wait()
    for g in range(GROUPS):
        o = g * L
        acc = bv
        for f in range(6):
            acc = acc + vals_v[0, pl.ds(f * BPW + o, L)]
        for k in range(F_CONT):
            acc = acc + (vals_v[0, pl.ds((6 + k) * BPW + o, L)]
                         * xv[0, pl.ds((F_IDX + k) * BPW + o, L)].astype(jnp.float32))
        out_v[pl.ds(o, L)] = acc
    pltpu.sync_copy(out_v, out_hbm.at[pl.ds(base, BPW)])


def kernel(x, fc_weight, bias):
    xt = x.astype(jnp.int32).T
    out = _fl_kernel(xt, fc_weight.astype(jnp.float32).T,
                     bias.astype(jnp.float32))
    return out.reshape(B, 1)


# speedup vs baseline: 3.1209x; 1.0198x over previous
import functools

import jax
import jax.numpy as jnp
from jax import lax
from jax.experimental import pallas as pl
from jax.experimental.pallas import tpu as pltpu
from jax.experimental.pallas import tpu_sc as plsc

B = 16384
NFLD = 12
F_IDX = 9
F_CONT = 3
NC = 2
NS = 16
L = 16
NW = NC * NS
BPW = B // NW
GROUPS = BPW // L

_mesh = plsc.VectorSubcoreMesh(core_axis_name="c", subcore_axis_name="s")


@functools.partial(
    pl.kernel,
    mesh=_mesh,
    compiler_params=pltpu.CompilerParams(needs_layout_passes=False),
    out_type=jax.ShapeDtypeStruct((B,), jnp.float32),
    scratch_types=[
        pltpu.VMEM((1, NFLD * BPW), jnp.int32),
        pltpu.VMEM((1, F_IDX * BPW), jnp.float32),
    pltpu.VMEM((BPW,), jnp.float32),
        pltpu.VMEM((L,), jnp.float32),
        pltpu.SemaphoreType.DMA,
        [pltpu.SemaphoreType.DMA] * NFLD,
    ],
)
def _fl_kernel(xt_hbm, table_hbm, bias_hbm, out_hbm,
               xv, vals_v, out_v, bias_v, sem, row_sems):
    wid = lax.axis_index("s") * NC + lax.axis_index("c")
    base = wid * BPW
    row_copies = [
        pltpu.async_copy(xt_hbm.at[pl.ds(f, 1), pl.ds(base, BPW)],
                         xv.at[:, pl.ds(f * BPW, BPW)], row_sems[f])
        for f in range(NFLD)
    ]
    copies = []
    for f in range(F_IDX):
        row_copies[f].wait()
        copies.append(
            pltpu.async_copy(table_hbm.at[xv.at[:, pl.ds(f * BPW, BPW)]],
                             vals_v.at[:, pl.ds(f * BPW, BPW)], sem))
    pltpu.sync_copy(bias_hbm, bias_v.at[pl.ds(0, 1)])
    for f in range(F_IDX, NFLD):
        row_copies[f].wait()
    lanes = lax.iota(jnp.int32, L)
    bv = plsc.load_gather(bias_v, [lanes * 0])
    for c in copies:
        c.wait()
    for g in range(GROUPS):
        o = g * L
        acc = bv
        for f in range(6):
            acc = acc + vals_v[0, pl.ds(f * BPW + o, L)]
        for k in range(F_CONT):
            acc = acc + (vals_v[0, pl.ds((6 + k) * BPW + o, L)]
                         * xv[0, pl.ds((F_IDX + k) * BPW + o, L)].astype(jnp.float32))
        out_v[pl.ds(o, L)] = acc
    pltpu.sync_copy(out_v, out_hbm.at[pl.ds(base, BPW)])


def kernel(x, fc_weight, bias):
    xt = x.astype(jnp.int32).T
    out = _fl_kernel(xt, fc_weight.astype(jnp.float32).T,
                     bias.astype(jnp.float32))
    return out.reshape(B, 1)
